# Initial kernel scaffold; baseline (speedup 1.0000x reference)
import jax
import jax.numpy as jnp
from jax.experimental import pallas as pl


def kernel(positions, edge_idx):
    raise NotImplementedError("write your pallas kernel here")


# speedup vs baseline: 10.6463x; 10.6463x over previous
import functools

import jax
import jax.numpy as jnp
from jax import lax
from jax.experimental import pallas as pl
from jax.experimental.pallas import tpu as pltpu
from jax.experimental.pallas import tpu_sc as plsc

NC = 2
NS = 16
NW = NC * NS
LANES = 16

CHUNK = 8000


def _rsqrt_newton(x):
    i = plsc.bitcast(x, jnp.int32)
    y = plsc.bitcast(jnp.int32(0x5F3759DF) - (i >> 1), jnp.float32)
    for _ in range(3):
        y = y * (1.5 - 0.5 * x * y * y)
    return y


def _make_sc_kernel(n_nodes, n_edges):
    per_w = n_edges // NW
    assert per_w * NW == n_edges and per_w % CHUNK == 0
    n_chunks = per_w // CHUNK
    groups = CHUNK // LANES

    mesh = plsc.VectorSubcoreMesh(core_axis_name="c", subcore_axis_name="s")

    @functools.partial(
        pl.kernel,
        mesh=mesh,
        compiler_params=pltpu.CompilerParams(needs_layout_passes=False),
        out_type=[
            jax.ShapeDtypeStruct((3 * n_edges,), jnp.float32),
            jax.ShapeDtypeStruct((n_edges,), jnp.float32),
        ],
        scratch_types=[
            pltpu.VMEM_SHARED((n_nodes,), jnp.float32),
            pltpu.VMEM_SHARED((n_nodes,), jnp.float32),
            pltpu.VMEM_SHARED((n_nodes,), jnp.float32),
            pltpu.VMEM((CHUNK,), jnp.int32),
            pltpu.VMEM((CHUNK,), jnp.int32),
            pltpu.VMEM((CHUNK,), jnp.float32),
            pltpu.VMEM((CHUNK,), jnp.float32),
            pltpu.VMEM((CHUNK,), jnp.float32),
            pltpu.VMEM((CHUNK,), jnp.float32),
            pltpu.VMEM((CHUNK,), jnp.float32),
            pltpu.VMEM((CHUNK,), jnp.float32),
            pltpu.VMEM((3 * CHUNK,), jnp.float32),
            pltpu.VMEM((CHUNK,), jnp.float32),
            pltpu.SemaphoreType.DMA,
        ],
    )
    def sc_kernel(px_hbm, py_hbm, pz_hbm, src_hbm, dst_hbm, diff_hbm, dist_hbm,
                  px_sh, py_sh, pz_sh, src_v, dst_v,
                  sx_v, sy_v, sz_v, dx_v, dy_v, dz_v, diff_v, dist_v, sem):
        cid = lax.axis_index("c")
        sid = lax.axis_index("s")
        wid = sid * NC + cid
        edge0 = wid * per_w

        @pl.when(sid == 0)
        def _stage():
            pltpu.sync_copy(px_hbm, px_sh)
            pltpu.sync_copy(py_hbm, py_sh)
            pltpu.sync_copy(pz_hbm, pz_sh)

        plsc.subcore_barrier()

        def chunk_body(k, carry):
            base = pl.multiple_of(edge0 + k * CHUNK, 8)
            pltpu.sync_copy(src_hbm.at[pl.ds(base, CHUNK)], src_v)
            pltpu.sync_copy(dst_hbm.at[pl.ds(base, CHUNK)], dst_v)
            pltpu.async_copy(px_sh.at[src_v], sx_v, sem)
            pltpu.async_copy(py_sh.at[src_v], sy_v, sem)
            pltpu.async_copy(pz_sh.at[src_v], sz_v, sem)
            pltpu.async_copy(px_sh.at[dst_v], dx_v, sem)
            pltpu.async_copy(py_sh.at[dst_v], dy_v, sem)
            cp = pltpu.async_copy(pz_sh.at[dst_v], dz_v, sem)
            for _ in range(6):
                cp.wait()

            def group_body(g, carry2):
                o = pl.multiple_of(g * LANES, LANES)
                lane = lax.iota(jnp.int32, LANES) + g * LANES
                ddx = dx_v[pl.ds(o, LANES)] - sx_v[pl.ds(o, LANES)]
                ddy = dy_v[pl.ds(o, LANES)] - sy_v[pl.ds(o, LANES)]
                ddz = dz_v[pl.ds(o, LANES)] - sz_v[pl.ds(o, LANES)]
                lane3 = lane * 3
                plsc.store_scatter(diff_v, [lane3], ddx)
                plsc.store_scatter(diff_v, [lane3 + 1], ddy)
                plsc.store_scatter(diff_v, [lane3 + 2], ddz)
                x = ddx * ddx + ddy * ddy + ddz * ddz
                xc = jnp.maximum(x, 1e-30)
                dist_v[pl.ds(o, LANES)] = xc * _rsqrt_newton(xc)
                return carry2

            lax.fori_loop(0, groups, group_body, 0)
            pltpu.sync_copy(diff_v, diff_hbm.at[pl.ds(3 * base, 3 * CHUNK)])
            pltpu.sync_copy(dist_v, dist_hbm.at[pl.ds(base, CHUNK)])
            return carry

        lax.fori_loop(0, n_chunks, chunk_body, 0)

    return sc_kernel


def kernel(positions, edge_idx):
    n_nodes = positions.shape[0]
    n_edges = edge_idx.shape[0]
    px = positions[:, 0]
    py = positions[:, 1]
    pz = positions[:, 2]
    src = edge_idx[:, 0]
    dst = edge_idx[:, 1]
    diff_flat, dist = _make_sc_kernel(n_nodes, n_edges)(px, py, pz, src, dst)
    return diff_flat.reshape(n_edges, 3), dist
